# Initial kernel scaffold; baseline (speedup 1.0000x reference)
import jax
import jax.numpy as jnp
from jax.experimental import pallas as pl


def kernel(x, codebook):
    raise NotImplementedError("write your pallas kernel here")


# speedup vs baseline: 1.6063x; 1.6063x over previous
import jax
import jax.numpy as jnp
from jax.experimental import pallas as pl

_KCODES = 8192
_DIM = 256
_TN = 256


def _vq_body(x_ref, cb_ref, q_ref, idx_ref, oh_ref):
    x = x_ref[...]
    cb = cb_ref[...]
    a2 = jnp.sum(x * x, axis=1, keepdims=True)
    b2 = jnp.sum(cb * cb, axis=1)[None, :]
    ab = jax.lax.dot_general(
        x, cb, (((1,), (1,)), ((), ())),
        preferred_element_type=jnp.float32)
    d = (a2 + b2) - 2.0 * ab
    m = jnp.min(d, axis=1, keepdims=True)
    iota = jax.lax.broadcasted_iota(jnp.int32, d.shape, 1)
    loc = jnp.min(jnp.where(d == m, iota, jnp.int32(_KCODES)),
                  axis=1, keepdims=True)
    oh = (iota == loc).astype(jnp.float32)
    oh_ref[...] = oh
    idx_ref[...] = loc
    q_ref[...] = jax.lax.dot_general(
        oh, cb, (((1,), (0,)), ((), ())),
        preferred_element_type=jnp.float32)


def kernel(x, codebook):
    b, t, d = x.shape
    n = b * t
    xf = x.reshape(n, d)
    q, idx, oh = pl.pallas_call(
        _vq_body,
        grid=(n // _TN,),
        in_specs=[
            pl.BlockSpec((_TN, d), lambda i: (i, 0)),
            pl.BlockSpec((_KCODES, d), lambda i: (0, 0)),
        ],
        out_specs=[
            pl.BlockSpec((_TN, d), lambda i: (i, 0)),
            pl.BlockSpec((_TN, 1), lambda i: (i, 0)),
            pl.BlockSpec((_TN, _KCODES), lambda i: (i, 0)),
        ],
        out_shape=[
            jax.ShapeDtypeStruct((n, d), jnp.float32),
            jax.ShapeDtypeStruct((n, 1), jnp.int32),
            jax.ShapeDtypeStruct((n, _KCODES), jnp.float32),
        ],
    )(xf, codebook)
    return (q.reshape(b, t, d), idx.reshape(b, t), oh.reshape(b, t, _KCODES))
